# baseline (speedup 1.0000x reference)
import jax
import jax.numpy as jnp
from jax.experimental import pallas as pl
from jax.experimental.pallas import tpu as pltpu

_D = 3
_N = 256
_H1 = 32
_H2 = 64


def _layer(h_T, W, b_col, d_out, use_relu):
    d_x = h_T.shape[0]
    WL = W[:, :d_x]
    WR = W[:, d_x:]
    f32 = jnp.float32

    G = jax.lax.dot_general(h_T, h_T, (((0,), (0,)), ((), ())),
                            preferred_element_type=f32)
    sq_row = jnp.sum(h_T * h_T, axis=0, keepdims=True)
    dred = sq_row - 2.0 * G
    minv = jnp.min(dred, axis=1, keepdims=True)
    lane = jax.lax.broadcasted_iota(jnp.int32, (_N, _N), 1)
    m_col = jnp.min(jnp.where(dred == minv, lane, _N),
                    axis=1, keepdims=True)
    P = (lane == m_col).astype(f32)

    A_T = jax.lax.dot_general(WL, h_T, (((1,), (0,)), ((), ())),
                              preferred_element_type=f32) + b_col
    C_T = jax.lax.dot_general(WR, h_T, (((1,), (0,)), ((), ())),
                              preferred_element_type=f32)
    Cm_T = jax.lax.dot_general(C_T, P, (((1,), (1,)), ((), ())),
                               preferred_element_type=f32)
    inv = f32(1.0 / (_N - 1))

    if not use_relu:
        sumC = jnp.sum(C_T, axis=1, keepdims=True)
        return A_T + (sumC - Cm_T) * inv

    C_nat = jax.lax.dot_general(h_T, WR, (((0,), (1,)), ((), ())),
                                preferred_element_type=f32)
    rows = []
    for k in range(d_out):
        col_c = C_nat[:, k:k + 1]
        row_a = A_T[k:k + 1, :]
        rk = jnp.maximum(col_c + row_a, 0.0)
        rows.append(jnp.sum(rk, axis=0, keepdims=True))
    S_T = jnp.concatenate(rows, axis=0)
    return (S_T - jnp.maximum(A_T + Cm_T, 0.0)) * inv


_BATCHES_PER_PROGRAM = 8


def _net_kernel(x_ref, W1_ref, b1_ref, W2_ref, b2_ref, W3_ref, b3_ref, out_ref):
    for i in range(_BATCHES_PER_PROGRAM):
        h = x_ref[i]
        h = _layer(h, W1_ref[...], b1_ref[...], _H1, True)
        h = _layer(h, W2_ref[...], b2_ref[...], _H2, True)
        h = _layer(h, W3_ref[...], b3_ref[...], _D, False)
        out_ref[i] = h


def kernel(x, W1, b1, W2, b2, W3, b3):
    B = x.shape[0]
    bpp = _BATCHES_PER_PROGRAM
    x_T = x.reshape(B, _N, _D).transpose(0, 2, 1)
    out = pl.pallas_call(
        _net_kernel,
        grid=(1,),
        in_specs=[
            pl.BlockSpec((bpp, _D, _N), lambda b: (0, 0, 0)),
            pl.BlockSpec(W1.shape, lambda b: (0, 0)),
            pl.BlockSpec((_H1, 1), lambda b: (0, 0)),
            pl.BlockSpec(W2.shape, lambda b: (0, 0)),
            pl.BlockSpec((_H2, 1), lambda b: (0, 0)),
            pl.BlockSpec(W3.shape, lambda b: (0, 0)),
            pl.BlockSpec((_D, 1), lambda b: (0, 0)),
        ],
        out_specs=pl.BlockSpec((bpp, _D, _N), lambda b: (0, 0, 0)),
        out_shape=jax.ShapeDtypeStruct((B, _D, _N), jnp.float32),
        compiler_params=pltpu.CompilerParams(
            dimension_semantics=("parallel",)),
    )(x_T, W1, b1.reshape(_H1, 1), W2, b2.reshape(_H2, 1),
      W3, b3.reshape(_D, 1))
    return out.transpose(0, 2, 1).reshape(B, _N * _D)


# speedup vs baseline: 1.3440x; 1.3440x over previous
import jax
import jax.numpy as jnp
from jax.experimental import pallas as pl
from jax.experimental.pallas import tpu as pltpu

_D = 3
_N = 256
_H1 = 32
_H2 = 64
_INV = 1.0 / (_N - 1)


def _nearest_onehot(G, sq_row):
    dred = sq_row - 2.0 * G
    minv = jnp.min(dred, axis=1, keepdims=True)
    lane = jax.lax.broadcasted_iota(jnp.int32, (_N, _N), 1)
    m_col = jnp.min(jnp.where(dred == minv, lane, _N),
                    axis=1, keepdims=True)
    return (lane == m_col).astype(jnp.float32)


def _layer(h, W, b_row, d_out, use_relu):
    f32 = jnp.float32
    d_x = h.shape[1]
    WL = W[:, :d_x]
    WR = W[:, d_x:]

    G = jax.lax.dot_general(h, h, (((1,), (1,)), ((), ())),
                            preferred_element_type=f32)
    eye = (jax.lax.broadcasted_iota(jnp.int32, (_N, _N), 0) ==
           jax.lax.broadcasted_iota(jnp.int32, (_N, _N), 1))
    sq_row = jnp.sum(jnp.where(eye, G, 0.0), axis=0, keepdims=True)
    P = _nearest_onehot(G, sq_row)

    A = jax.lax.dot_general(h, WL, (((1,), (1,)), ((), ())),
                            preferred_element_type=f32) + b_row
    C = jax.lax.dot_general(h, WR, (((1,), (1,)), ((), ())),
                            preferred_element_type=f32)
    Cm = jax.lax.dot_general(P, C, (((1,), (0,)), ((), ())),
                             preferred_element_type=f32)

    if not use_relu:
        sumC = jnp.sum(C, axis=0, keepdims=True)
        return A + (sumC - Cm) * _INV

    p = 128 // d_out
    nb = _N // p
    Cpack = jnp.concatenate([C[s * nb:(s + 1) * nb, :] for s in range(p)],
                            axis=1)
    Apack = jnp.concatenate([A] * p, axis=1)
    acc = jnp.maximum(Apack + Cpack[0:1, :], 0.0)
    for jj in range(1, nb):
        acc = acc + jnp.maximum(Apack + Cpack[jj:jj + 1, :], 0.0)
    S = acc[:, :d_out]
    for s in range(1, p):
        S = S + acc[:, s * d_out:(s + 1) * d_out]
    return (S - jnp.maximum(A + Cm, 0.0)) * _INV


_BPP = 8


def _net_kernel(x_ref, W1_ref, b1_ref, W2_ref, b2_ref, W3_ref, b3_ref, out_ref):
    for i in range(_BPP):
        h = x_ref[i]
        h = _layer(h, W1_ref[...], b1_ref[...], _H1, True)
        h = _layer(h, W2_ref[...], b2_ref[...], _H2, True)
        h = _layer(h, W3_ref[...], b3_ref[...], _D, False)
        out_ref[i] = h


def kernel(x, W1, b1, W2, b2, W3, b3):
    B = x.shape[0]
    x3 = x.reshape(B, _N, _D)
    out = pl.pallas_call(
        _net_kernel,
        grid=(1,),
        in_specs=[
            pl.BlockSpec((_BPP, _N, _D), lambda b: (0, 0, 0)),
            pl.BlockSpec(W1.shape, lambda b: (0, 0)),
            pl.BlockSpec((1, _H1), lambda b: (0, 0)),
            pl.BlockSpec(W2.shape, lambda b: (0, 0)),
            pl.BlockSpec((1, _H2), lambda b: (0, 0)),
            pl.BlockSpec(W3.shape, lambda b: (0, 0)),
            pl.BlockSpec((1, _D), lambda b: (0, 0)),
        ],
        out_specs=pl.BlockSpec((_BPP, _N, _D), lambda b: (0, 0, 0)),
        out_shape=jax.ShapeDtypeStruct((B, _N, _D), jnp.float32),
        compiler_params=pltpu.CompilerParams(
            dimension_semantics=("arbitrary",)),
    )(x3, W1, b1.reshape(1, _H1), W2, b2.reshape(1, _H2),
      W3, b3.reshape(1, _D))
    return out.reshape(B, _N * _D)
